# Initial kernel scaffold; baseline (speedup 1.0000x reference)
import jax
import jax.numpy as jnp
from jax.experimental import pallas as pl


def kernel(x, edge_index, edge_attr, batch, params):
    raise NotImplementedError("write your pallas kernel here")


# speedup vs baseline: 1.8398x; 1.8398x over previous
import functools

import jax
import jax.numpy as jnp
from jax import lax
from jax.experimental import pallas as pl
from jax.experimental.pallas import tpu as pltpu
from jax.experimental.pallas import tpu_sc as plsc

N = 50000
E = 800000
H = 64
NG = 8

NC, NS, L = 2, 16, 16
NP = 50176
NHALF = NP // 2
ACC = 25600
DUM = ACC - 1
EPT = E // NS
KC = 2000
CH = 80
BN = 1024
BE = 1280


def _enc_body(xb, w1, b1, w2, b2, g, b, ob):
    h = jnp.maximum(jnp.dot(xb[...], w1[...], preferred_element_type=jnp.float32)
                    + b1[...], 0.0)
    h = jnp.dot(h, w2[...], preferred_element_type=jnp.float32) + b2[...]
    m = jnp.mean(h, axis=-1, keepdims=True)
    v = jnp.mean((h - m) * (h - m), axis=-1, keepdims=True)
    ob[...] = (h - m) * lax.rsqrt(v + 1e-5) * g[...] + b[...]


def _encoder(xp, w1, b1, w2, b2, g, b, blk):
    rows = xp.shape[0]
    grid = rows // blk
    full = lambda shape: pl.BlockSpec(shape, lambda i: (0, 0))
    return pl.pallas_call(
        _enc_body,
        grid=(grid,),
        in_specs=[pl.BlockSpec((blk, 8), lambda i: (i, 0)),
                  full((8, H)), full((1, H)), full((H, H)), full((1, H)),
                  full((1, H)), full((1, H))],
        out_specs=pl.BlockSpec((blk, H), lambda i: (i, 0)),
        out_shape=jax.ShapeDtypeStruct((rows, H), jnp.float32),
    )(xp, w1, b1, w2, b2, g, b)


def _gin_body(hb, ab, eps, w1, b1, w2, b2, g, b, ob, *, relu_out):
    z = hb[...] * (1.0 + eps[...]) + ab[...]
    z = jnp.maximum(jnp.dot(z, w1[...], preferred_element_type=jnp.float32)
                    + b1[...], 0.0)
    z = jnp.dot(z, w2[...], preferred_element_type=jnp.float32) + b2[...]
    m = jnp.mean(z, axis=-1, keepdims=True)
    v = jnp.mean((z - m) * (z - m), axis=-1, keepdims=True)
    z = (z - m) * lax.rsqrt(v + 1e-5) * g[...] + b[...]
    if relu_out:
        z = jnp.maximum(z, 0.0)
    ob[...] = z


def _gin_mlp(h, aggr, eps, w1, b1, w2, b2, g, b, relu_out):
    grid = NP // BN
    full = lambda shape: pl.BlockSpec(shape, lambda i: (0, 0))
    return pl.pallas_call(
        functools.partial(_gin_body, relu_out=relu_out),
        grid=(grid,),
        in_specs=[pl.BlockSpec((BN, H), lambda i: (i, 0)),
                  pl.BlockSpec((BN, H), lambda i: (i, 0)),
                  full((1, 1)), full((H, H)), full((1, H)), full((H, H)),
                  full((1, H)), full((1, H)), full((1, H))],
        out_specs=pl.BlockSpec((BN, H), lambda i: (i, 0)),
        out_shape=jax.ShapeDtypeStruct((NP, H), jnp.float32),
    )(h, aggr, eps, w1, b1, w2, b2, g, b)


def _pool_body(hb, bb, gw, gbias, gg, gbeta, gout, sout, accv, cntv):
    i = pl.program_id(0)
    ng = pl.num_programs(0)

    @pl.when(i == 0)
    def _init():
        accv[...] = jnp.zeros_like(accv)
        cntv[...] = jnp.zeros_like(cntv)

    ridx = i * BN + lax.broadcasted_iota(jnp.int32, (BN, 1), 0)
    hclean = jnp.where(ridx < N, hb[...], 0.0)
    bvec = bb[0, 0, :]
    oh = (bvec[None, :] == lax.broadcasted_iota(jnp.int32, (NG, BN), 0)
          ).astype(jnp.float32)
    accv[...] += jnp.dot(oh, hclean, preferred_element_type=jnp.float32)
    cntv[...] += jnp.sum(oh, axis=1, keepdims=True)

    @pl.when(i == ng - 1)
    def _fin():
        counts = cntv[:, 0:1]
        gmean = accv[...] / jnp.maximum(counts, 1.0)
        t = jnp.maximum(jnp.dot(gmean, gw[...],
                                preferred_element_type=jnp.float32)
                        + gbias[...], 0.0)
        m = jnp.mean(t, axis=-1, keepdims=True)
        v = jnp.mean((t - m) * (t - m), axis=-1, keepdims=True)
        gout[...] = (t - m) * lax.rsqrt(v + 1e-5) * gg[...] + gbeta[...]
        kk = lax.broadcasted_iota(jnp.int32, (NG, NG), 0)
        jj = lax.broadcasted_iota(jnp.int32, (NG, NG), 1)
        mask = (jj < kk).astype(jnp.float32)
        starts = jnp.sum(mask * counts[None, :, 0], axis=1)
        sout[...] = starts[None, :].astype(jnp.int32)


def _pool(h, batch_r, gw, gbias, gg, gbeta):
    grid = NP // BN
    full = lambda shape: pl.BlockSpec(shape, lambda i: (0, 0))
    return pl.pallas_call(
        _pool_body,
        grid=(grid,),
        in_specs=[pl.BlockSpec((BN, H), lambda i: (i, 0)),
                  pl.BlockSpec((1, 1, BN), lambda i: (i, 0, 0)),
                  full((H, H)), full((1, H)), full((1, H)), full((1, H))],
        out_specs=[full((NG, H)), full((1, NG))],
        out_shape=[jax.ShapeDtypeStruct((NG, H), jnp.float32),
                   jax.ShapeDtypeStruct((1, NG), jnp.int32)],
        scratch_shapes=[pltpu.VMEM((NG, H), jnp.float32),
                        pltpu.VMEM((NG, 1), jnp.float32)],
    )(h, batch_r, gw, gbias, gg, gbeta)


def _edge_body(hsb, hdb, efb, srcb, starts, g, w1a, w1b, w1c, w1d, b1,
               w2, b2, w3, b3, ob):
    sv = srcb[0, 0, :]
    st = starts[0, :]
    ge = (sv[:, None] >= st[None, :]).astype(jnp.int32)
    bs = jnp.sum(ge, axis=1) - 1
    oh = (bs[:, None] ==
          lax.broadcasted_iota(jnp.int32, (BE, NG), 1)).astype(jnp.float32)
    g2 = jnp.dot(g[...], w1c[...], preferred_element_type=jnp.float32)
    t = (jnp.dot(hsb[...], w1a[...], preferred_element_type=jnp.float32)
         + jnp.dot(hdb[...], w1b[...], preferred_element_type=jnp.float32)
         + jnp.dot(efb[...], w1d[...], preferred_element_type=jnp.float32)
         + jnp.dot(oh, g2, preferred_element_type=jnp.float32)
         + b1[...])
    t = jnp.tanh(t)
    t = jnp.tanh(jnp.dot(t, w2[...], preferred_element_type=jnp.float32)
                 + b2[...])
    t = jnp.dot(t, w3[...], preferred_element_type=jnp.float32) + b3[...]
    ob[...] = 1.0 / (1.0 + jnp.exp(-t))


def _edge_mlp(hs, hd, ef, src_r, starts, g, w1a, w1b, w1c, w1d, b1, w2, b2,
              w3, b3):
    grid = E // BE
    full = lambda shape: pl.BlockSpec(shape, lambda i: (0,) * len(shape))
    return pl.pallas_call(
        _edge_body,
        grid=(grid,),
        in_specs=[pl.BlockSpec((BE, H), lambda i: (i, 0)),
                  pl.BlockSpec((BE, H), lambda i: (i, 0)),
                  pl.BlockSpec((BE, H), lambda i: (i, 0)),
                  pl.BlockSpec((1, 1, BE), lambda i: (i, 0, 0)),
                  full((1, NG)), full((NG, H)),
                  full((H, 2 * H)), full((H, 2 * H)), full((H, 2 * H)),
                  full((H, 2 * H)), full((1, 2 * H)),
                  full((2 * H, H)), full((1, H)), full((H, 1)), full((1, 1))],
        out_specs=pl.BlockSpec((BE, 1), lambda i: (i, 0)),
        out_shape=jax.ShapeDtypeStruct((E, 1), jnp.float32),
    )(hs, hd, ef, src_r, starts, g, w1a, w1b, w1c, w1d, b1, w2, b2, w3, b3)


def _sc_aggr_body(h_hbm, ef_hbm, src_hbm, dst_hbm, out_hbm,
                  sidx, didx, lidx, rows, efb, accum):
    c = lax.axis_index("c")
    s = lax.axis_index("s")
    base = c * NHALF

    @pl.loop(0, CH)
    def _zr(i):
        for j in range(H // L):
            rows[i, pl.ds(j * L, L)] = jnp.zeros((L,), jnp.float32)

    rows_per_tec = ACC // NS
    @pl.loop(0, rows_per_tec // CH)
    def _zc(k):
        pltpu.sync_copy(rows, accum.at[pl.ds(s * rows_per_tec + k * CH, CH)])

    plsc.subcore_barrier()

    e0 = s * EPT
    @pl.loop(0, EPT // KC)
    def _outer(ko):
        off = e0 + ko * KC
        pltpu.sync_copy(src_hbm.at[pl.ds(off, KC)], sidx)
        pltpu.sync_copy(dst_hbm.at[pl.ds(off, KC)], didx)

        @pl.loop(0, KC // CH)
        def _lix(r):
            for j in range(CH // L):
                v = didx[pl.ds(r * CH + j * L, L)] - base
                oob = (v < 0) | (v >= NHALF)
                lidx[r, pl.ds(j * L, L)] = jnp.where(oob, DUM, v)

        @pl.loop(0, KC // CH)
        def _inner(jc):
            eoff = off + jc * CH
            pltpu.sync_copy(h_hbm.at[sidx.at[pl.ds(jc * CH, CH)]], rows)
            pltpu.sync_copy(ef_hbm.at[pl.ds(eoff, CH)], efb)

            @pl.loop(0, CH)
            def _cmp(i):
                for j in range(H // L):
                    rows[i, pl.ds(j * L, L)] = jnp.maximum(
                        rows[i, pl.ds(j * L, L)] + efb[i, pl.ds(j * L, L)],
                        0.0)

            pltpu.sync_copy(rows, accum.at[lidx.at[jc]], add=True)

    plsc.subcore_barrier()

    per_tec = NHALF // NS
    a0 = s * per_tec
    @pl.loop(0, per_tec // CH)
    def _co(k):
        a = a0 + k * CH
        pltpu.sync_copy(accum.at[pl.ds(a, CH)], rows)
        pltpu.sync_copy(rows, out_hbm.at[c].at[pl.ds(a, CH)])

    rem = per_tec % CH
    a = a0 + (per_tec // CH) * CH
    pltpu.sync_copy(accum.at[pl.ds(a, rem)], rows.at[pl.ds(0, rem)])
    pltpu.sync_copy(rows.at[pl.ds(0, rem)], out_hbm.at[c].at[pl.ds(a, rem)])


def _sc_aggr(h, ef, src, dst):
    mesh = plsc.VectorSubcoreMesh(core_axis_name="c", subcore_axis_name="s")
    f = pl.kernel(
        _sc_aggr_body,
        out_type=jax.ShapeDtypeStruct((NC, NHALF, H), jnp.float32),
        mesh=mesh,
        scratch_types=[pltpu.VMEM((KC,), jnp.int32),
                       pltpu.VMEM((KC,), jnp.int32),
                       pltpu.VMEM((KC // CH, CH), jnp.int32),
                       pltpu.VMEM((CH, H), jnp.float32),
                       pltpu.VMEM((CH, H), jnp.float32),
                       pltpu.VMEM_SHARED((ACC, H), jnp.float32)],
        compiler_params=pltpu.CompilerParams(use_tc_tiling_on_sc=False),
    )
    return f(h, ef, src, dst)


def _sc_gather2_body(h_hbm, src_hbm, dst_hbm, hs_hbm, hd_hbm,
                     sidx, rows):
    c = lax.axis_index("c")
    s = lax.axis_index("s")
    e0 = s * EPT

    def run(idx_hbm, out_hbm):
        @pl.loop(0, EPT // KC)
        def _outer(ko):
            off = e0 + ko * KC
            pltpu.sync_copy(idx_hbm.at[pl.ds(off, KC)], sidx)

            @pl.loop(0, KC // CH)
            def _inner(jc):
                pltpu.sync_copy(h_hbm.at[sidx.at[pl.ds(jc * CH, CH)]], rows)
                pltpu.sync_copy(rows, out_hbm.at[pl.ds(off + jc * CH, CH)])

    @pl.when(c == 0)
    def _c0():
        run(src_hbm, hs_hbm)

    @pl.when(c == 1)
    def _c1():
        run(dst_hbm, hd_hbm)


def _sc_gather2(h, src, dst):
    mesh = plsc.VectorSubcoreMesh(core_axis_name="c", subcore_axis_name="s")
    f = pl.kernel(
        _sc_gather2_body,
        out_type=[jax.ShapeDtypeStruct((E, H), jnp.float32),
                  jax.ShapeDtypeStruct((E, H), jnp.float32)],
        mesh=mesh,
        scratch_types=[pltpu.VMEM((KC,), jnp.int32),
                       pltpu.VMEM((CH, H), jnp.float32)],
        compiler_params=pltpu.CompilerParams(use_tc_tiling_on_sc=False),
    )
    return f(h, src, dst)


def kernel(x, edge_index, edge_attr, batch, params):
    p = params
    src = edge_index[0]
    dst = edge_index[1]

    xp = jnp.pad(x, ((0, NP - N), (0, 5)))
    eap = jnp.pad(edge_attr, ((0, 0), (0, 5)))
    batch_r = jnp.pad(batch, (0, NP - N), constant_values=NG).reshape(
        NP // BN, 1, BN)
    src_r = src.reshape(E // BE, 1, BE)

    r1 = lambda a: a.reshape(1, -1)
    ne_w1 = jnp.pad(p['ne_w1'], ((0, 5), (0, 0)))
    ee_w1 = jnp.pad(p['ee_w1'], ((0, 5), (0, 0)))

    h = _encoder(xp, ne_w1, r1(p['ne_b1']), p['ne_w2'], r1(p['ne_b2']),
                 r1(p['ne_g']), r1(p['ne_b']), BN)
    ef = _encoder(eap, ee_w1, r1(p['ee_b1']), p['ee_w2'], r1(p['ee_b2']),
                  r1(p['ee_g']), r1(p['ee_b']), BE)

    for i in range(2):
        q = p['gin%d' % i]
        aggr = _sc_aggr(h, ef, src, dst).reshape(NP, H)
        h = _gin_mlp(h, aggr, q['eps'].reshape(1, 1), q['w1'], r1(q['b1']),
                     q['w2'], r1(q['b2']), r1(q['g']), r1(q['b']),
                     relu_out=(i == 0))

    g, starts = _pool(h, batch_r, p['gp_w'], r1(p['gp_b']), r1(p['gp_g']),
                      r1(p['gp_beta']))

    hs, hd = _sc_gather2(h, src, dst)

    w1 = p['ep_w1']
    o = _edge_mlp(hs, hd, ef, src_r, starts, g,
                  w1[0:H], w1[H:2 * H], w1[2 * H:3 * H], w1[3 * H:4 * H],
                  r1(p['ep_b1']), p['ep_w2'], r1(p['ep_b2']), p['ep_w3'],
                  p['ep_b3'].reshape(1, 1))
    return o


# speedup vs baseline: 2.0647x; 1.1223x over previous
import functools

import jax
import jax.numpy as jnp
from jax import lax
from jax.experimental import pallas as pl
from jax.experimental.pallas import tpu as pltpu
from jax.experimental.pallas import tpu_sc as plsc

N = 50000
E = 800000
H = 64
H2 = 128
NG = 8

NC, NS, L = 2, 16, 16
NW = NC * NS
NP = 50176
NHALF = NP // 2
ACC = 25104
DUM = ACC - 1
EPW = E // NW
CH1 = 40
KC1 = 1000
EPT = E // NS
CH2 = 80
KC2 = 2000
BN = 1024
BE = 1280


def _masked_ln(z, g, b):
    mask = (lax.broadcasted_iota(jnp.int32, (1, H2), 1) < H).astype(jnp.float32)
    m = jnp.sum(z, axis=-1, keepdims=True) * (1.0 / H)
    zc = (z - m) * mask
    v = jnp.sum(zc * zc, axis=-1, keepdims=True) * (1.0 / H)
    return zc * lax.rsqrt(v + 1e-5) * g + b


def _enc_body(xb, w1, b1, w2, b2, g, b, ob, *, wide):
    h = jnp.maximum(jnp.dot(xb[...], w1[...], preferred_element_type=jnp.float32)
                    + b1[...], 0.0)
    z = jnp.dot(h, w2[...], preferred_element_type=jnp.float32) + b2[...]
    if wide:
        ob[...] = _masked_ln(z, g[...], b[...])
    else:
        m = jnp.mean(z, axis=-1, keepdims=True)
        v = jnp.mean((z - m) * (z - m), axis=-1, keepdims=True)
        ob[...] = (z - m) * lax.rsqrt(v + 1e-5) * g[...] + b[...]


def _encoder(xp, w1, b1, w2, b2, g, b, blk, wide):
    rows = xp.shape[0]
    grid = rows // blk
    hout = H2 if wide else H
    full = lambda shape: pl.BlockSpec(shape, lambda i: (0, 0))
    return pl.pallas_call(
        functools.partial(_enc_body, wide=wide),
        grid=(grid,),
        in_specs=[pl.BlockSpec((blk, 8), lambda i: (i, 0)),
                  full((8, H)), full((1, H)), full((H, hout)), full((1, hout)),
                  full((1, hout)), full((1, hout))],
        out_specs=pl.BlockSpec((blk, hout), lambda i: (i, 0)),
        out_shape=jax.ShapeDtypeStruct((rows, hout), jnp.float32),
    )(xp, w1, b1, w2, b2, g, b)


def _gin_body(hb, ab, eps, w1, b1, w2, b2, g, b, ob, *, relu_out):
    z = hb[:, 0:H] * (1.0 + eps[...]) + ab[:, 0:H]
    z = jnp.maximum(jnp.dot(z, w1[...], preferred_element_type=jnp.float32)
                    + b1[...], 0.0)
    z = jnp.dot(z, w2[...], preferred_element_type=jnp.float32) + b2[...]
    z = _masked_ln(z, g[...], b[...])
    if relu_out:
        z = jnp.maximum(z, 0.0)
    ob[...] = z


def _gin_mlp(h, aggr, eps, w1, b1, w2, b2, g, b, relu_out):
    grid = NP // BN
    full = lambda shape: pl.BlockSpec(shape, lambda i: (0, 0))
    return pl.pallas_call(
        functools.partial(_gin_body, relu_out=relu_out),
        grid=(grid,),
        in_specs=[pl.BlockSpec((BN, H2), lambda i: (i, 0)),
                  pl.BlockSpec((BN, H), lambda i: (i, 0)),
                  full((1, 1)), full((H, H)), full((1, H)), full((H, H2)),
                  full((1, H2)), full((1, H2)), full((1, H2))],
        out_specs=pl.BlockSpec((BN, H2), lambda i: (i, 0)),
        out_shape=jax.ShapeDtypeStruct((NP, H2), jnp.float32),
    )(h, aggr, eps, w1, b1, w2, b2, g, b)


def _pool_body(hb, bb, gw, gbias, gg, gbeta, gout, sout, accv, cntv):
    i = pl.program_id(0)
    ng = pl.num_programs(0)

    @pl.when(i == 0)
    def _init():
        accv[...] = jnp.zeros_like(accv)
        cntv[...] = jnp.zeros_like(cntv)

    ridx = i * BN + lax.broadcasted_iota(jnp.int32, (BN, 1), 0)
    hclean = jnp.where(ridx < N, hb[:, 0:H], 0.0)
    bvec = bb[0, 0, :]
    oh = (bvec[None, :] == lax.broadcasted_iota(jnp.int32, (NG, BN), 0)
          ).astype(jnp.float32)
    accv[...] += jnp.dot(oh, hclean, preferred_element_type=jnp.float32)
    cntv[...] += jnp.sum(oh, axis=1, keepdims=True)

    @pl.when(i == ng - 1)
    def _fin():
        counts = cntv[:, 0:1]
        gmean = accv[...] / jnp.maximum(counts, 1.0)
        t = jnp.maximum(jnp.dot(gmean, gw[...],
                                preferred_element_type=jnp.float32)
                        + gbias[...], 0.0)
        m = jnp.mean(t, axis=-1, keepdims=True)
        v = jnp.mean((t - m) * (t - m), axis=-1, keepdims=True)
        gout[...] = (t - m) * lax.rsqrt(v + 1e-5) * gg[...] + gbeta[...]
        kk = lax.broadcasted_iota(jnp.int32, (NG, NG), 0)
        jj = lax.broadcasted_iota(jnp.int32, (NG, NG), 1)
        mask = (jj < kk).astype(jnp.float32)
        starts = jnp.sum(mask * counts[None, :, 0], axis=1)
        sout[...] = starts[None, :].astype(jnp.int32)


def _pool(h, batch_r, gw, gbias, gg, gbeta):
    grid = NP // BN
    full = lambda shape: pl.BlockSpec(shape, lambda i: (0, 0))
    return pl.pallas_call(
        _pool_body,
        grid=(grid,),
        in_specs=[pl.BlockSpec((BN, H2), lambda i: (i, 0)),
                  pl.BlockSpec((1, 1, BN), lambda i: (i, 0, 0)),
                  full((H, H)), full((1, H)), full((1, H)), full((1, H))],
        out_specs=[full((NG, H)), full((1, NG))],
        out_shape=[jax.ShapeDtypeStruct((NG, H), jnp.float32),
                   jax.ShapeDtypeStruct((1, NG), jnp.int32)],
        scratch_shapes=[pltpu.VMEM((NG, H), jnp.float32),
                        pltpu.VMEM((NG, 1), jnp.float32)],
    )(h, batch_r, gw, gbias, gg, gbeta)


def _edge_body(hsb, hdb, efb, srcb, starts, g, w1a, w1b, w1c, w1d, b1,
               w2, b2, w3, b3, ob):
    sv = srcb[0, 0, :]
    st = starts[0, :]
    ge = (sv[:, None] >= st[None, :]).astype(jnp.int32)
    bs = jnp.sum(ge, axis=1) - 1
    oh = (bs[:, None] ==
          lax.broadcasted_iota(jnp.int32, (BE, NG), 1)).astype(jnp.float32)
    g2 = jnp.dot(g[...], w1c[...], preferred_element_type=jnp.float32)
    t = (jnp.dot(hsb[...], w1a[...], preferred_element_type=jnp.float32)
         + jnp.dot(hdb[...], w1b[...], preferred_element_type=jnp.float32)
         + jnp.dot(efb[...], w1d[...], preferred_element_type=jnp.float32)
         + jnp.dot(oh, g2, preferred_element_type=jnp.float32)
         + b1[...])
    t = jnp.tanh(t)
    t = jnp.tanh(jnp.dot(t, w2[...], preferred_element_type=jnp.float32)
                 + b2[...])
    t = jnp.dot(t, w3[...], preferred_element_type=jnp.float32) + b3[...]
    ob[...] = 1.0 / (1.0 + jnp.exp(-t))


def _edge_mlp(hs, hd, ef, src_r, starts, g, w1a, w1b, w1c, w1d, b1, w2, b2,
              w3, b3):
    grid = E // BE
    full = lambda shape: pl.BlockSpec(shape, lambda i: (0,) * len(shape))
    return pl.pallas_call(
        _edge_body,
        grid=(grid,),
        in_specs=[pl.BlockSpec((BE, H), lambda i: (i, 0)),
                  pl.BlockSpec((BE, H), lambda i: (i, 0)),
                  pl.BlockSpec((BE, H), lambda i: (i, 0)),
                  pl.BlockSpec((1, 1, BE), lambda i: (i, 0, 0)),
                  full((1, NG)), full((NG, H)),
                  full((H, 2 * H)), full((H, 2 * H)), full((H, 2 * H)),
                  full((H, 2 * H)), full((1, 2 * H)),
                  full((2 * H, H)), full((1, H)), full((H, 1)), full((1, 1))],
        out_specs=pl.BlockSpec((BE, 1), lambda i: (i, 0)),
        out_shape=jax.ShapeDtypeStruct((E, 1), jnp.float32),
    )(hs, hd, ef, src_r, starts, g, w1a, w1b, w1c, w1d, b1, w2, b2, w3, b3)


def _sc_msg_body(h_hbm, ef_hbm, idx_hbm, out_hbm, sidx, rows0, rows1,
                 efb0, efb1, msg0, msg1, sg0, sg1, se0, se1, sw0, sw1,
                 *, with_ef):
    c = lax.axis_index("c")
    sub = lax.axis_index("s")
    wid = sub * NC + c
    e0 = wid * EPW
    NCHUNK = KC1 // CH1

    def start_gather(jc, rows, sg):
        pltpu.async_copy(h_hbm.at[sidx.at[pl.ds(jc * CH1, CH1)]], rows, sg)

    def start_ef(goff, efb, se):
        if with_ef:
            pltpu.async_copy(ef_hbm.at[pl.ds(goff, CH1)], efb, se)

    def wait_g(rows, sem):
        pltpu.make_async_copy(h_hbm.at[sidx.at[pl.ds(0, CH1)]], rows,
                              sem).wait()

    def wait_e(efb, sem):
        pltpu.make_async_copy(ef_hbm.at[pl.ds(0, CH1)], efb, sem).wait()

    def wait_w(msg, sem):
        pltpu.make_async_copy(msg, out_hbm.at[pl.ds(0, CH1)], sem).wait()

    def compute(rows, efb, msg):
        @pl.loop(0, CH1)
        def _cmp(i):
            for j in range(H // L):
                v = rows[i, pl.ds(j * L, L)]
                if with_ef:
                    v = jnp.maximum(v + efb[i, pl.ds(j * L, L)], 0.0)
                msg[i, pl.ds(j * L, L)] = v

    @pl.loop(0, EPW // KC1)
    def _outer(ko):
        off = e0 + ko * KC1
        pltpu.sync_copy(idx_hbm.at[pl.ds(off, KC1)], sidx)
        start_gather(0, rows0, sg0)
        start_ef(off, efb0, se0)

        @pl.loop(0, NCHUNK)
        def _inner(jc):
            goff = pl.multiple_of(off + jc * CH1, 8)
            even = jc % 2 == 0

            @pl.when((jc + 1 < NCHUNK) & even)
            def _s1():
                start_gather(jc + 1, rows1, sg1)
                start_ef(goff + CH1, efb1, se1)

            @pl.when((jc + 1 < NCHUNK) & (~even))
            def _s0():
                start_gather(jc + 1, rows0, sg0)
                start_ef(goff + CH1, efb0, se0)

            @pl.when(even)
            def _c0():
                wait_g(rows0, sg0)
                if with_ef:
                    wait_e(efb0, se0)
                @pl.when(jc >= 2)
                def _wb():
                    wait_w(msg0, sw0)
                compute(rows0, efb0, msg0)
                pltpu.async_copy(msg0, out_hbm.at[pl.ds(goff, CH1)], sw0)

            @pl.when(~even)
            def _c1():
                wait_g(rows1, sg1)
                if with_ef:
                    wait_e(efb1, se1)
                @pl.when(jc >= 2)
                def _wb():
                    wait_w(msg1, sw1)
                compute(rows1, efb1, msg1)
                pltpu.async_copy(msg1, out_hbm.at[pl.ds(goff, CH1)], sw1)

        wait_w(msg1, sw1)
        wait_w(msg0, sw0)


def _sc_msg(h, ef, idx, with_ef):
    mesh = plsc.VectorSubcoreMesh(core_axis_name="c", subcore_axis_name="s")
    sems = [pltpu.SemaphoreType.DMA] * 6
    f = pl.kernel(
        functools.partial(_sc_msg_body, with_ef=with_ef),
        out_type=jax.ShapeDtypeStruct((E, H), jnp.float32),
        mesh=mesh,
        scratch_types=[pltpu.VMEM((KC1,), jnp.int32),
                       pltpu.VMEM((CH1, H2), jnp.float32),
                       pltpu.VMEM((CH1, H2), jnp.float32),
                       pltpu.VMEM((CH1, H), jnp.float32),
                       pltpu.VMEM((CH1, H), jnp.float32),
                       pltpu.VMEM((CH1, H), jnp.float32),
                       pltpu.VMEM((CH1, H), jnp.float32)] + sems,
    )
    return f(h, ef, idx)


def _sc_seg_body(msg_hbm, dst_hbm, out_hbm, didx, lidx, mb0, mb1,
                 accum, sr0, sr1):
    c = lax.axis_index("c")
    sub = lax.axis_index("s")
    base = c * NHALF
    NCHUNK = KC2 // CH2

    @pl.loop(0, CH2)
    def _zr(i):
        for j in range(H // L):
            mb0[i, pl.ds(j * L, L)] = jnp.zeros((L,), jnp.float32)

    zpt = ACC // NS
    z0 = sub * zpt
    @pl.loop(0, zpt // CH2)
    def _zc(k):
        pltpu.sync_copy(mb0, accum.at[pl.ds(z0 + k * CH2, CH2)])
    pltpu.sync_copy(mb0.at[pl.ds(0, zpt % CH2)],
                    accum.at[pl.ds(z0 + (zpt // CH2) * CH2, zpt % CH2)])

    plsc.subcore_barrier()

    def wait_read(ref, sem):
        pltpu.make_async_copy(msg_hbm.at[pl.ds(0, CH2)], ref, sem).wait()

    e0 = sub * EPT
    @pl.loop(0, EPT // KC2)
    def _outer(ko):
        off = e0 + ko * KC2
        pltpu.sync_copy(dst_hbm.at[pl.ds(off, KC2)], didx)

        @pl.loop(0, NCHUNK)
        def _lix(r):
            for j in range(CH2 // L):
                v = didx[pl.ds(r * CH2 + j * L, L)] - base
                oob = (v < 0) | (v >= NHALF)
                lidx[r, pl.ds(j * L, L)] = jnp.where(oob, DUM, v)

        pltpu.async_copy(msg_hbm.at[pl.ds(pl.multiple_of(off, 8), CH2)],
                         mb0, sr0)

        @pl.loop(0, NCHUNK)
        def _inner(jc):
            goff = pl.multiple_of(off + jc * CH2, 8)
            even = jc % 2 == 0

            @pl.when((jc + 1 < NCHUNK) & even)
            def _s1():
                pltpu.async_copy(msg_hbm.at[pl.ds(goff + CH2, CH2)], mb1, sr1)

            @pl.when((jc + 1 < NCHUNK) & (~even))
            def _s0():
                pltpu.async_copy(msg_hbm.at[pl.ds(goff + CH2, CH2)], mb0, sr0)

            @pl.when(even)
            def _c0():
                wait_read(mb0, sr0)
                pltpu.sync_copy(mb0, accum.at[lidx.at[jc]], add=True)

            @pl.when(~even)
            def _c1():
                wait_read(mb1, sr1)
                pltpu.sync_copy(mb1, accum.at[lidx.at[jc]], add=True)

    plsc.subcore_barrier()

    per_tec = NHALF // NS
    a0 = sub * per_tec
    @pl.loop(0, per_tec // CH2)
    def _co(k):
        a = pl.multiple_of(a0 + k * CH2, 8)
        pltpu.sync_copy(accum.at[pl.ds(a, CH2)], mb0)
        pltpu.sync_copy(mb0, out_hbm.at[pl.ds(pl.multiple_of(
            base + a, 8), CH2)])

    rem = per_tec % CH2
    a = pl.multiple_of(a0 + (per_tec // CH2) * CH2, 8)
    pltpu.sync_copy(accum.at[pl.ds(a, rem)], mb0.at[pl.ds(0, rem)])
    pltpu.sync_copy(mb0.at[pl.ds(0, rem)],
                    out_hbm.at[pl.ds(pl.multiple_of(base + a, 8), rem)])


def _sc_seg(msg, dst):
    mesh = plsc.VectorSubcoreMesh(core_axis_name="c", subcore_axis_name="s")
    f = pl.kernel(
        _sc_seg_body,
        out_type=jax.ShapeDtypeStruct((NP, H), jnp.float32),
        mesh=mesh,
        scratch_types=[pltpu.VMEM((KC2,), jnp.int32),
                       pltpu.VMEM((KC2 // CH2, CH2), jnp.int32),
                       pltpu.VMEM((CH2, H), jnp.float32),
                       pltpu.VMEM((CH2, H), jnp.float32),
                       pltpu.VMEM_SHARED((ACC, H), jnp.float32),
                       pltpu.SemaphoreType.DMA, pltpu.SemaphoreType.DMA],
        compiler_params=pltpu.CompilerParams(use_tc_tiling_on_sc=False),
    )
    return f(msg, dst)
def kernel(x, edge_index, edge_attr, batch, params):
    p = params
    src = edge_index[0]
    dst = edge_index[1]

    xp = jnp.pad(x, ((0, NP - N), (0, 5)))
    eap = jnp.pad(edge_attr, ((0, 0), (0, 5)))
    batch_r = jnp.pad(batch, (0, NP - N), constant_values=NG).reshape(
        NP // BN, 1, BN)
    src_r = src.reshape(E // BE, 1, BE)

    r1 = lambda a: a.reshape(1, -1)
    padw = lambda a: jnp.pad(a, ((0, 0), (0, H2 - H)))
    ne_w1 = jnp.pad(p['ne_w1'], ((0, 5), (0, 0)))
    ee_w1 = jnp.pad(p['ee_w1'], ((0, 5), (0, 0)))

    h = _encoder(xp, ne_w1, r1(p['ne_b1']), padw(p['ne_w2']),
                 padw(r1(p['ne_b2'])), padw(r1(p['ne_g'])),
                 padw(r1(p['ne_b'])), BN, wide=True)
    ef = _encoder(eap, ee_w1, r1(p['ee_b1']), p['ee_w2'], r1(p['ee_b2']),
                  r1(p['ee_g']), r1(p['ee_b']), BE, wide=False)

    for i in range(2):
        q = p['gin%d' % i]
        msg = _sc_msg(h, ef, src, with_ef=True)
        aggr = _sc_seg(msg, dst)
        h = _gin_mlp(h, aggr, q['eps'].reshape(1, 1), q['w1'], r1(q['b1']),
                     padw(q['w2']), padw(r1(q['b2'])), padw(r1(q['g'])),
                     padw(r1(q['b'])), relu_out=(i == 0))

    g, starts = _pool(h, batch_r, p['gp_w'], r1(p['gp_b']), r1(p['gp_g']),
                      r1(p['gp_beta']))

    hs = _sc_msg(h, ef, src, with_ef=False)
    hd = _sc_msg(h, ef, dst, with_ef=False)

    w1 = p['ep_w1']
    o = _edge_mlp(hs, hd, ef, src_r, starts, g,
                  w1[0:H], w1[H:2 * H], w1[2 * H:3 * H], w1[3 * H:4 * H],
                  r1(p['ep_b1']), p['ep_w2'], r1(p['ep_b2']), p['ep_w3'],
                  p['ep_b3'].reshape(1, 1))
    return o


# speedup vs baseline: 2.1250x; 1.0292x over previous
import functools

import jax
import jax.numpy as jnp
from jax import lax
from jax.experimental import pallas as pl
from jax.experimental.pallas import tpu as pltpu
from jax.experimental.pallas import tpu_sc as plsc

N = 50000
E = 800000
H = 64
H2 = 128
NG = 8

NC, NS, L = 2, 16, 16
NW = NC * NS
NP = 50176
NHALF = NP // 2
ACC = 25104
DUM = ACC - 1
EPW = E // NW
CH1 = 40
KC1 = 1000
EPT = E // NS
CH2 = 80
KC2 = 2000
BN = 1024
BE = 1280


def _masked_ln(z, g, b):
    mask = (lax.broadcasted_iota(jnp.int32, (1, H2), 1) < H).astype(jnp.float32)
    m = jnp.sum(z, axis=-1, keepdims=True) * (1.0 / H)
    zc = (z - m) * mask
    v = jnp.sum(zc * zc, axis=-1, keepdims=True) * (1.0 / H)
    return zc * lax.rsqrt(v + 1e-5) * g + b


def _enc_body(xb, w1, b1, w2, b2, g, b, ob, *, wide):
    h = jnp.maximum(jnp.dot(xb[...], w1[...], preferred_element_type=jnp.float32)
                    + b1[...], 0.0)
    z = jnp.dot(h, w2[...], preferred_element_type=jnp.float32) + b2[...]
    if wide:
        ob[...] = _masked_ln(z, g[...], b[...])
    else:
        m = jnp.mean(z, axis=-1, keepdims=True)
        v = jnp.mean((z - m) * (z - m), axis=-1, keepdims=True)
        ob[...] = (z - m) * lax.rsqrt(v + 1e-5) * g[...] + b[...]


def _encoder(xp, w1, b1, w2, b2, g, b, blk, wide):
    rows = xp.shape[0]
    grid = rows // blk
    hout = H2 if wide else H
    full = lambda shape: pl.BlockSpec(shape, lambda i: (0, 0))
    return pl.pallas_call(
        functools.partial(_enc_body, wide=wide),
        grid=(grid,),
        in_specs=[pl.BlockSpec((blk, 8), lambda i: (i, 0)),
                  full((8, H)), full((1, H)), full((H, hout)), full((1, hout)),
                  full((1, hout)), full((1, hout))],
        out_specs=pl.BlockSpec((blk, hout), lambda i: (i, 0)),
        out_shape=jax.ShapeDtypeStruct((rows, hout), jnp.float32),
    )(xp, w1, b1, w2, b2, g, b)


def _gin_body(hb, ab, eps, w1, b1, w2, b2, g, b, ob, *, relu_out):
    z = hb[:, 0:H] * (1.0 + eps[...]) + ab[:, 0:H]
    z = jnp.maximum(jnp.dot(z, w1[...], preferred_element_type=jnp.float32)
                    + b1[...], 0.0)
    z = jnp.dot(z, w2[...], preferred_element_type=jnp.float32) + b2[...]
    z = _masked_ln(z, g[...], b[...])
    if relu_out:
        z = jnp.maximum(z, 0.0)
    ob[...] = z


def _gin_mlp(h, aggr, eps, w1, b1, w2, b2, g, b, relu_out):
    grid = NP // BN
    full = lambda shape: pl.BlockSpec(shape, lambda i: (0, 0))
    return pl.pallas_call(
        functools.partial(_gin_body, relu_out=relu_out),
        grid=(grid,),
        in_specs=[pl.BlockSpec((BN, H2), lambda i: (i, 0)),
                  pl.BlockSpec((BN, H), lambda i: (i, 0)),
                  full((1, 1)), full((H, H)), full((1, H)), full((H, H2)),
                  full((1, H2)), full((1, H2)), full((1, H2))],
        out_specs=pl.BlockSpec((BN, H2), lambda i: (i, 0)),
        out_shape=jax.ShapeDtypeStruct((NP, H2), jnp.float32),
    )(h, aggr, eps, w1, b1, w2, b2, g, b)


def _pool_body(hb, bb, gw, gbias, gg, gbeta, gout, sout, accv, cntv):
    i = pl.program_id(0)
    ng = pl.num_programs(0)

    @pl.when(i == 0)
    def _init():
        accv[...] = jnp.zeros_like(accv)
        cntv[...] = jnp.zeros_like(cntv)

    ridx = i * BN + lax.broadcasted_iota(jnp.int32, (BN, 1), 0)
    hclean = jnp.where(ridx < N, hb[:, 0:H], 0.0)
    bvec = bb[0, 0, :]
    oh = (bvec[None, :] == lax.broadcasted_iota(jnp.int32, (NG, BN), 0)
          ).astype(jnp.float32)
    accv[...] += jnp.dot(oh, hclean, preferred_element_type=jnp.float32)
    cntv[...] += jnp.sum(oh, axis=1, keepdims=True)

    @pl.when(i == ng - 1)
    def _fin():
        counts = cntv[:, 0:1]
        gmean = accv[...] / jnp.maximum(counts, 1.0)
        t = jnp.maximum(jnp.dot(gmean, gw[...],
                                preferred_element_type=jnp.float32)
                        + gbias[...], 0.0)
        m = jnp.mean(t, axis=-1, keepdims=True)
        v = jnp.mean((t - m) * (t - m), axis=-1, keepdims=True)
        gout[...] = (t - m) * lax.rsqrt(v + 1e-5) * gg[...] + gbeta[...]
        kk = lax.broadcasted_iota(jnp.int32, (NG, NG), 0)
        jj = lax.broadcasted_iota(jnp.int32, (NG, NG), 1)
        mask = (jj < kk).astype(jnp.float32)
        starts = jnp.sum(mask * counts[None, :, 0], axis=1)
        sout[...] = starts[None, :].astype(jnp.int32)


def _pool(h, batch_r, gw, gbias, gg, gbeta):
    grid = NP // BN
    full = lambda shape: pl.BlockSpec(shape, lambda i: (0, 0))
    return pl.pallas_call(
        _pool_body,
        grid=(grid,),
        in_specs=[pl.BlockSpec((BN, H2), lambda i: (i, 0)),
                  pl.BlockSpec((1, 1, BN), lambda i: (i, 0, 0)),
                  full((H, H)), full((1, H)), full((1, H)), full((1, H))],
        out_specs=[full((NG, H)), full((1, NG))],
        out_shape=[jax.ShapeDtypeStruct((NG, H), jnp.float32),
                   jax.ShapeDtypeStruct((1, NG), jnp.int32)],
        scratch_shapes=[pltpu.VMEM((NG, H), jnp.float32),
                        pltpu.VMEM((NG, 1), jnp.float32)],
    )(h, batch_r, gw, gbias, gg, gbeta)


def _edge_body(hsb, hdb, efb, srcb, starts, g, w1a, w1b, w1c, w1d, b1,
               w2, b2, w3, b3, ob):
    sv = srcb[0, 0, :]
    st = starts[0, :]
    ge = (sv[:, None] >= st[None, :]).astype(jnp.int32)
    bs = jnp.sum(ge, axis=1) - 1
    oh = (bs[:, None] ==
          lax.broadcasted_iota(jnp.int32, (BE, NG), 1)).astype(jnp.float32)
    g2 = jnp.dot(g[...], w1c[...], preferred_element_type=jnp.float32)
    t = (jnp.dot(hsb[...], w1a[...], preferred_element_type=jnp.float32)
         + jnp.dot(hdb[...], w1b[...], preferred_element_type=jnp.float32)
         + jnp.dot(efb[...], w1d[...], preferred_element_type=jnp.float32)
         + jnp.dot(oh, g2, preferred_element_type=jnp.float32)
         + b1[...])
    t = jnp.tanh(t)
    t = jnp.tanh(jnp.dot(t, w2[...], preferred_element_type=jnp.float32)
                 + b2[...])
    t = jnp.dot(t, w3[...], preferred_element_type=jnp.float32) + b3[...]
    ob[...] = 1.0 / (1.0 + jnp.exp(-t))


def _edge_mlp(hs, hd, ef, src_r, starts, g, w1a, w1b, w1c, w1d, b1, w2, b2,
              w3, b3):
    grid = E // BE
    full = lambda shape: pl.BlockSpec(shape, lambda i: (0,) * len(shape))
    return pl.pallas_call(
        _edge_body,
        grid=(grid,),
        in_specs=[pl.BlockSpec((BE, H), lambda i: (i, 0)),
                  pl.BlockSpec((BE, H), lambda i: (i, 0)),
                  pl.BlockSpec((BE, H), lambda i: (i, 0)),
                  pl.BlockSpec((1, 1, BE), lambda i: (i, 0, 0)),
                  full((1, NG)), full((NG, H)),
                  full((H, 2 * H)), full((H, 2 * H)), full((H, 2 * H)),
                  full((H, 2 * H)), full((1, 2 * H)),
                  full((2 * H, H)), full((1, H)), full((H, 1)), full((1, 1))],
        out_specs=pl.BlockSpec((BE, 1), lambda i: (i, 0)),
        out_shape=jax.ShapeDtypeStruct((E, 1), jnp.float32),
    )(hs, hd, ef, src_r, starts, g, w1a, w1b, w1c, w1d, b1, w2, b2, w3, b3)


def _sc_msg_body(h_hbm, ef_hbm, idx_hbm, out_hbm, sidx, rows0, rows1,
                 efb0, efb1, msg0, msg1, sg0, sg1, se0, se1, sw0, sw1,
                 *, with_ef, out_1d):
    c = lax.axis_index("c")
    sub = lax.axis_index("s")
    wid = sub * NC + c
    e0 = wid * EPW
    NCHUNK = KC1 // CH1

    def start_gather(jc, rows, sg):
        pltpu.async_copy(h_hbm.at[sidx.at[pl.ds(jc * CH1, CH1)]], rows, sg)

    def start_ef(goff, efb, se):
        if with_ef:
            pltpu.async_copy(ef_hbm.at[pl.ds(goff, CH1)], efb, se)

    def wait_g(rows, sem):
        pltpu.make_async_copy(h_hbm.at[sidx.at[pl.ds(0, CH1)]], rows,
                              sem).wait()

    def wait_e(efb, sem):
        pltpu.make_async_copy(ef_hbm.at[pl.ds(0, CH1)], efb, sem).wait()

    def wait_w(msg, sem):
        n = CH1 * H if out_1d else CH1
        pltpu.make_async_copy(msg, out_hbm.at[pl.ds(0, n)], sem).wait()

    def start_w(msg, goff, sw):
        if out_1d:
            pltpu.async_copy(
                msg, out_hbm.at[pl.ds(pl.multiple_of(goff * H, 8), CH1 * H)],
                sw)
        else:
            pltpu.async_copy(msg, out_hbm.at[pl.ds(goff, CH1)], sw)

    def compute(rows, efb, msg):
        @pl.loop(0, CH1)
        def _cmp(i):
            for j in range(H // L):
                v = rows[i, pl.ds(j * L, L)]
                if with_ef:
                    v = jnp.maximum(v + efb[i, pl.ds(j * L, L)], 0.0)
                if out_1d:
                    msg[pl.ds(i * H + j * L, L)] = v
                else:
                    msg[i, pl.ds(j * L, L)] = v

    @pl.loop(0, EPW // KC1)
    def _outer(ko):
        off = e0 + ko * KC1
        pltpu.sync_copy(idx_hbm.at[pl.ds(off, KC1)], sidx)
        start_gather(0, rows0, sg0)
        start_ef(off, efb0, se0)

        @pl.loop(0, NCHUNK)
        def _inner(jc):
            goff = pl.multiple_of(off + jc * CH1, 8)
            even = jc % 2 == 0

            @pl.when((jc + 1 < NCHUNK) & even)
            def _s1():
                start_gather(jc + 1, rows1, sg1)
                start_ef(goff + CH1, efb1, se1)

            @pl.when((jc + 1 < NCHUNK) & (~even))
            def _s0():
                start_gather(jc + 1, rows0, sg0)
                start_ef(goff + CH1, efb0, se0)

            @pl.when(even)
            def _c0():
                wait_g(rows0, sg0)
                if with_ef:
                    wait_e(efb0, se0)
                @pl.when(jc >= 2)
                def _wb():
                    wait_w(msg0, sw0)
                compute(rows0, efb0, msg0)
                start_w(msg0, goff, sw0)

            @pl.when(~even)
            def _c1():
                wait_g(rows1, sg1)
                if with_ef:
                    wait_e(efb1, se1)
                @pl.when(jc >= 2)
                def _wb():
                    wait_w(msg1, sw1)
                compute(rows1, efb1, msg1)
                start_w(msg1, goff, sw1)

        wait_w(msg1, sw1)
        wait_w(msg0, sw0)


def _sc_msg(h, ef, idx, with_ef, out_1d=False):
    mesh = plsc.VectorSubcoreMesh(core_axis_name="c", subcore_axis_name="s")
    sems = [pltpu.SemaphoreType.DMA] * 6
    oshape = (jax.ShapeDtypeStruct((E * H,), jnp.float32) if out_1d
              else jax.ShapeDtypeStruct((E, H), jnp.float32))
    mshape = ((CH1 * H,) if out_1d else (CH1, H))
    f = pl.kernel(
        functools.partial(_sc_msg_body, with_ef=with_ef, out_1d=out_1d),
        out_type=oshape,
        mesh=mesh,
        scratch_types=[pltpu.VMEM((KC1,), jnp.int32),
                       pltpu.VMEM((CH1, H2), jnp.float32),
                       pltpu.VMEM((CH1, H2), jnp.float32),
                       pltpu.VMEM((CH1, H), jnp.float32),
                       pltpu.VMEM((CH1, H), jnp.float32),
                       pltpu.VMEM(mshape, jnp.float32),
                       pltpu.VMEM(mshape, jnp.float32)] + sems,
    )
    return f(h, ef, idx)


def _sc_seg_body(msg_hbm, dst_hbm, out_hbm, didx, lidx, mb0, mb1, mbc,
                 accum, sr0, sr1):
    c = lax.axis_index("c")
    sub = lax.axis_index("s")
    base = c * NHALF
    NCHUNK = KC2 // CH2

    @pl.loop(0, CH2)
    def _zr(i):
        for j in range(H // L):
            mbc[i, pl.ds(j * L, L)] = jnp.zeros((L,), jnp.float32)

    zpt = ACC // NS
    z0 = sub * zpt
    @pl.loop(0, zpt // CH2)
    def _zc(k):
        pltpu.sync_copy(mbc, accum.at[pl.ds(z0 + k * CH2, CH2)])
    pltpu.sync_copy(mbc.at[pl.ds(0, zpt % CH2)],
                    accum.at[pl.ds(z0 + (zpt // CH2) * CH2, zpt % CH2)])

    plsc.subcore_barrier()

    def wait_read(ref, sem):
        pltpu.make_async_copy(msg_hbm.at[pl.ds(0, CH2 * H)], ref, sem).wait()

    def compact_scatter(mb, jc):
        @pl.loop(0, CH2)
        def _cp(i):
            for j in range(H // L):
                mbc[i, pl.ds(j * L, L)] = mb[pl.ds(i * H + j * L, L)]
        pltpu.sync_copy(mbc, accum.at[lidx.at[jc]], add=True)

    e0 = sub * EPT
    @pl.loop(0, EPT // KC2)
    def _outer(ko):
        off = e0 + ko * KC2
        pltpu.sync_copy(dst_hbm.at[pl.ds(off, KC2)], didx)

        @pl.loop(0, NCHUNK)
        def _lix(r):
            for j in range(CH2 // L):
                v = didx[pl.ds(r * CH2 + j * L, L)] - base
                oob = (v < 0) | (v >= NHALF)
                lidx[r, pl.ds(j * L, L)] = jnp.where(oob, DUM, v)

        pltpu.async_copy(msg_hbm.at[pl.ds(pl.multiple_of(off * H, 8),
                                          CH2 * H)], mb0, sr0)

        @pl.loop(0, NCHUNK)
        def _inner(jc):
            goff = pl.multiple_of((off + jc * CH2) * H, 8)
            even = jc % 2 == 0

            @pl.when((jc + 1 < NCHUNK) & even)
            def _s1():
                pltpu.async_copy(msg_hbm.at[pl.ds(goff + CH2 * H, CH2 * H)],
                                 mb1, sr1)

            @pl.when((jc + 1 < NCHUNK) & (~even))
            def _s0():
                pltpu.async_copy(msg_hbm.at[pl.ds(goff + CH2 * H, CH2 * H)],
                                 mb0, sr0)

            @pl.when(even)
            def _c0():
                wait_read(mb0, sr0)
                compact_scatter(mb0, jc)

            @pl.when(~even)
            def _c1():
                wait_read(mb1, sr1)
                compact_scatter(mb1, jc)

    plsc.subcore_barrier()

    per_tec = NHALF // NS
    a0 = sub * per_tec
    @pl.loop(0, per_tec // CH2)
    def _co(k):
        a = pl.multiple_of(a0 + k * CH2, 8)
        pltpu.sync_copy(accum.at[pl.ds(a, CH2)], mbc)
        pltpu.sync_copy(mbc, out_hbm.at[pl.ds(pl.multiple_of(
            base + a, 8), CH2)])

    rem = per_tec % CH2
    a = pl.multiple_of(a0 + (per_tec // CH2) * CH2, 8)
    pltpu.sync_copy(accum.at[pl.ds(a, rem)], mbc.at[pl.ds(0, rem)])
    pltpu.sync_copy(mbc.at[pl.ds(0, rem)],
                    out_hbm.at[pl.ds(pl.multiple_of(base + a, 8), rem)])


def _sc_seg(msg, dst):
    mesh = plsc.VectorSubcoreMesh(core_axis_name="c", subcore_axis_name="s")
    f = pl.kernel(
        _sc_seg_body,
        out_type=jax.ShapeDtypeStruct((NP, H), jnp.float32),
        mesh=mesh,
        scratch_types=[pltpu.VMEM((KC2,), jnp.int32),
                       pltpu.VMEM((KC2 // CH2, CH2), jnp.int32),
                       pltpu.VMEM((CH2 * H,), jnp.float32),
                       pltpu.VMEM((CH2 * H,), jnp.float32),
                       pltpu.VMEM((CH2, H), jnp.float32),
                       pltpu.VMEM_SHARED((ACC, H), jnp.float32),
                       pltpu.SemaphoreType.DMA, pltpu.SemaphoreType.DMA],
        compiler_params=pltpu.CompilerParams(use_tc_tiling_on_sc=False),
    )
    return f(msg, dst)
def kernel(x, edge_index, edge_attr, batch, params):
    p = params
    src = edge_index[0]
    dst = edge_index[1]

    xp = jnp.pad(x, ((0, NP - N), (0, 5)))
    eap = jnp.pad(edge_attr, ((0, 0), (0, 5)))
    batch_r = jnp.pad(batch, (0, NP - N), constant_values=NG).reshape(
        NP // BN, 1, BN)
    src_r = src.reshape(E // BE, 1, BE)

    r1 = lambda a: a.reshape(1, -1)
    padw = lambda a: jnp.pad(a, ((0, 0), (0, H2 - H)))
    ne_w1 = jnp.pad(p['ne_w1'], ((0, 5), (0, 0)))
    ee_w1 = jnp.pad(p['ee_w1'], ((0, 5), (0, 0)))

    h = _encoder(xp, ne_w1, r1(p['ne_b1']), padw(p['ne_w2']),
                 padw(r1(p['ne_b2'])), padw(r1(p['ne_g'])),
                 padw(r1(p['ne_b'])), BN, wide=True)
    ef = _encoder(eap, ee_w1, r1(p['ee_b1']), p['ee_w2'], r1(p['ee_b2']),
                  r1(p['ee_g']), r1(p['ee_b']), BE, wide=False)

    for i in range(2):
        q = p['gin%d' % i]
        msg = _sc_msg(h, ef, src, with_ef=True, out_1d=True)
        aggr = _sc_seg(msg, dst)
        h = _gin_mlp(h, aggr, q['eps'].reshape(1, 1), q['w1'], r1(q['b1']),
                     padw(q['w2']), padw(r1(q['b2'])), padw(r1(q['g'])),
                     padw(r1(q['b'])), relu_out=(i == 0))

    g, starts = _pool(h, batch_r, p['gp_w'], r1(p['gp_b']), r1(p['gp_g']),
                      r1(p['gp_beta']))

    hs = _sc_msg(h, ef, src, with_ef=False)
    hd = _sc_msg(h, ef, dst, with_ef=False)

    w1 = p['ep_w1']
    o = _edge_mlp(hs, hd, ef, src_r, starts, g,
                  w1[0:H], w1[H:2 * H], w1[2 * H:3 * H], w1[3 * H:4 * H],
                  r1(p['ep_b1']), p['ep_w2'], r1(p['ep_b2']), p['ep_w3'],
                  p['ep_b3'].reshape(1, 1))
    return o


# speedup vs baseline: 2.1255x; 1.0002x over previous
import functools

import jax
import jax.numpy as jnp
from jax import lax
from jax.experimental import pallas as pl
from jax.experimental.pallas import tpu as pltpu
from jax.experimental.pallas import tpu_sc as plsc

N = 50000
E = 800000
H = 64
H2 = 128
NG = 8

NC, NS, L = 2, 16, 16
NW = NC * NS
NP = 50176
NHALF = NP // 2
ACC = 25104
DUM = ACC - 1
EPW = E // NW
CH1 = 40
KC1 = 1000
EPT = E // NS
CH2 = 80
KC2 = 2000
BN = 1024
BE = 1280


def _masked_ln(z, g, b):
    mask = (lax.broadcasted_iota(jnp.int32, (1, H2), 1) < H).astype(jnp.float32)
    m = jnp.sum(z, axis=-1, keepdims=True) * (1.0 / H)
    zc = (z - m) * mask
    v = jnp.sum(zc * zc, axis=-1, keepdims=True) * (1.0 / H)
    return zc * lax.rsqrt(v + 1e-5) * g + b


def _enc_body(xb, w1, b1, w2, b2, g, b, ob, *, wide):
    h = jnp.maximum(jnp.dot(xb[...], w1[...], preferred_element_type=jnp.float32)
                    + b1[...], 0.0)
    z = jnp.dot(h, w2[...], preferred_element_type=jnp.float32) + b2[...]
    if wide:
        ob[...] = _masked_ln(z, g[...], b[...])
    else:
        m = jnp.mean(z, axis=-1, keepdims=True)
        v = jnp.mean((z - m) * (z - m), axis=-1, keepdims=True)
        ob[...] = (z - m) * lax.rsqrt(v + 1e-5) * g[...] + b[...]


def _encoder(xp, w1, b1, w2, b2, g, b, blk, wide):
    rows = xp.shape[0]
    grid = rows // blk
    hout = H2 if wide else H
    full = lambda shape: pl.BlockSpec(shape, lambda i: (0, 0))
    return pl.pallas_call(
        functools.partial(_enc_body, wide=wide),
        grid=(grid,),
        in_specs=[pl.BlockSpec((blk, 8), lambda i: (i, 0)),
                  full((8, H)), full((1, H)), full((H, hout)), full((1, hout)),
                  full((1, hout)), full((1, hout))],
        out_specs=pl.BlockSpec((blk, hout), lambda i: (i, 0)),
        out_shape=jax.ShapeDtypeStruct((rows, hout), jnp.float32),
    )(xp, w1, b1, w2, b2, g, b)


def _gin_body(hb, ab, eps, w1, b1, w2, b2, g, b, ob, *, relu_out):
    z = hb[:, 0:H] * (1.0 + eps[...]) + ab[:, 0:H]
    z = jnp.maximum(jnp.dot(z, w1[...], preferred_element_type=jnp.float32)
                    + b1[...], 0.0)
    z = jnp.dot(z, w2[...], preferred_element_type=jnp.float32) + b2[...]
    z = _masked_ln(z, g[...], b[...])
    if relu_out:
        z = jnp.maximum(z, 0.0)
    ob[...] = z


def _gin_mlp(h, aggr, eps, w1, b1, w2, b2, g, b, relu_out):
    grid = NP // BN
    full = lambda shape: pl.BlockSpec(shape, lambda i: (0, 0))
    return pl.pallas_call(
        functools.partial(_gin_body, relu_out=relu_out),
        grid=(grid,),
        in_specs=[pl.BlockSpec((BN, H2), lambda i: (i, 0)),
                  pl.BlockSpec((BN, H), lambda i: (i, 0)),
                  full((1, 1)), full((H, H)), full((1, H)), full((H, H2)),
                  full((1, H2)), full((1, H2)), full((1, H2))],
        out_specs=pl.BlockSpec((BN, H2), lambda i: (i, 0)),
        out_shape=jax.ShapeDtypeStruct((NP, H2), jnp.float32),
    )(h, aggr, eps, w1, b1, w2, b2, g, b)


def _pool_body(hb, bb, gw, gbias, gg, gbeta, gout, sout, accv, cntv):
    i = pl.program_id(0)
    ng = pl.num_programs(0)

    @pl.when(i == 0)
    def _init():
        accv[...] = jnp.zeros_like(accv)
        cntv[...] = jnp.zeros_like(cntv)

    ridx = i * BN + lax.broadcasted_iota(jnp.int32, (BN, 1), 0)
    hclean = jnp.where(ridx < N, hb[:, 0:H], 0.0)
    bvec = bb[0, 0, :]
    oh = (bvec[None, :] == lax.broadcasted_iota(jnp.int32, (NG, BN), 0)
          ).astype(jnp.float32)
    accv[...] += jnp.dot(oh, hclean, preferred_element_type=jnp.float32)
    cntv[...] += jnp.sum(oh, axis=1, keepdims=True)

    @pl.when(i == ng - 1)
    def _fin():
        counts = cntv[:, 0:1]
        gmean = accv[...] / jnp.maximum(counts, 1.0)
        t = jnp.maximum(jnp.dot(gmean, gw[...],
                                preferred_element_type=jnp.float32)
                        + gbias[...], 0.0)
        m = jnp.mean(t, axis=-1, keepdims=True)
        v = jnp.mean((t - m) * (t - m), axis=-1, keepdims=True)
        gout[...] = (t - m) * lax.rsqrt(v + 1e-5) * gg[...] + gbeta[...]
        kk = lax.broadcasted_iota(jnp.int32, (NG, NG), 0)
        jj = lax.broadcasted_iota(jnp.int32, (NG, NG), 1)
        mask = (jj < kk).astype(jnp.float32)
        starts = jnp.sum(mask * counts[None, :, 0], axis=1)
        sout[...] = starts[None, :].astype(jnp.int32)


def _pool(h, batch_r, gw, gbias, gg, gbeta):
    grid = NP // BN
    full = lambda shape: pl.BlockSpec(shape, lambda i: (0, 0))
    return pl.pallas_call(
        _pool_body,
        grid=(grid,),
        in_specs=[pl.BlockSpec((BN, H2), lambda i: (i, 0)),
                  pl.BlockSpec((1, 1, BN), lambda i: (i, 0, 0)),
                  full((H, H)), full((1, H)), full((1, H)), full((1, H))],
        out_specs=[full((NG, H)), full((1, NG))],
        out_shape=[jax.ShapeDtypeStruct((NG, H), jnp.float32),
                   jax.ShapeDtypeStruct((1, NG), jnp.int32)],
        scratch_shapes=[pltpu.VMEM((NG, H), jnp.float32),
                        pltpu.VMEM((NG, 1), jnp.float32)],
    )(h, batch_r, gw, gbias, gg, gbeta)


def _edge_body(hsb, hdb, efb, srcb, starts, g, w1a, w1b, w1c, w1d, b1,
               w2, b2, w3, b3, ob):
    sv = srcb[0, 0, :]
    st = starts[0, :]
    ge = (sv[:, None] >= st[None, :]).astype(jnp.int32)
    bs = jnp.sum(ge, axis=1) - 1
    oh = (bs[:, None] ==
          lax.broadcasted_iota(jnp.int32, (BE, NG), 1)).astype(jnp.float32)
    g2 = jnp.dot(g[...], w1c[...], preferred_element_type=jnp.float32)
    t = (jnp.dot(hsb[...], w1a[...], preferred_element_type=jnp.float32)
         + jnp.dot(hdb[...], w1b[...], preferred_element_type=jnp.float32)
         + jnp.dot(efb[...], w1d[...], preferred_element_type=jnp.float32)
         + jnp.dot(oh, g2, preferred_element_type=jnp.float32)
         + b1[...])
    t = jnp.tanh(t)
    t = jnp.tanh(jnp.dot(t, w2[...], preferred_element_type=jnp.float32)
                 + b2[...])
    t = jnp.dot(t, w3[...], preferred_element_type=jnp.float32) + b3[...]
    ob[...] = 1.0 / (1.0 + jnp.exp(-t))


def _edge_mlp(hs, hd, ef, src_r, starts, g, w1a, w1b, w1c, w1d, b1, w2, b2,
              w3, b3):
    grid = E // BE
    full = lambda shape: pl.BlockSpec(shape, lambda i: (0,) * len(shape))
    return pl.pallas_call(
        _edge_body,
        grid=(grid,),
        in_specs=[pl.BlockSpec((BE, H), lambda i: (i, 0)),
                  pl.BlockSpec((BE, H), lambda i: (i, 0)),
                  pl.BlockSpec((BE, H), lambda i: (i, 0)),
                  pl.BlockSpec((1, 1, BE), lambda i: (i, 0, 0)),
                  full((1, NG)), full((NG, H)),
                  full((H, 2 * H)), full((H, 2 * H)), full((H, 2 * H)),
                  full((H, 2 * H)), full((1, 2 * H)),
                  full((2 * H, H)), full((1, H)), full((H, 1)), full((1, 1))],
        out_specs=pl.BlockSpec((BE, 1), lambda i: (i, 0)),
        out_shape=jax.ShapeDtypeStruct((E, 1), jnp.float32),
    )(hs, hd, ef, src_r, starts, g, w1a, w1b, w1c, w1d, b1, w2, b2, w3, b3)


def _sc_msg_body(h_hbm, ef_hbm, idx_hbm, out_hbm, sidx, rows0, rows1,
                 efb0, efb1, msg0, msg1, sg0, sg1, se0, se1, sw0, sw1,
                 *, with_ef, out_1d):
    c = lax.axis_index("c")
    sub = lax.axis_index("s")
    wid = sub * NC + c
    e0 = wid * EPW
    NCHUNK = KC1 // CH1

    def start_gather(jc, rows, sg):
        pltpu.async_copy(h_hbm.at[sidx.at[pl.ds(jc * CH1, CH1)]], rows, sg)

    def start_ef(goff, efb, se):
        if with_ef:
            pltpu.async_copy(ef_hbm.at[pl.ds(goff, CH1)], efb, se)

    def wait_g(rows, sem):
        pltpu.make_async_copy(h_hbm.at[sidx.at[pl.ds(0, CH1)]], rows,
                              sem).wait()

    def wait_e(efb, sem):
        pltpu.make_async_copy(ef_hbm.at[pl.ds(0, CH1)], efb, sem).wait()

    def wait_w(msg, sem):
        n = CH1 * H if out_1d else CH1
        pltpu.make_async_copy(msg, out_hbm.at[pl.ds(0, n)], sem).wait()

    def start_w(msg, goff, sw):
        if out_1d:
            pltpu.async_copy(
                msg, out_hbm.at[pl.ds(pl.multiple_of(goff * H, 8), CH1 * H)],
                sw)
        else:
            pltpu.async_copy(msg, out_hbm.at[pl.ds(goff, CH1)], sw)

    def compute(rows, efb, msg):
        @pl.loop(0, CH1)
        def _cmp(i):
            for j in range(H // L):
                v = rows[i, pl.ds(j * L, L)]
                if with_ef:
                    v = jnp.maximum(v + efb[i, pl.ds(j * L, L)], 0.0)
                if out_1d:
                    msg[pl.ds(i * H + j * L, L)] = v
                else:
                    msg[i, pl.ds(j * L, L)] = v

    @pl.loop(0, EPW // KC1)
    def _outer(ko):
        off = e0 + ko * KC1
        pltpu.sync_copy(idx_hbm.at[pl.ds(off, KC1)], sidx)
        start_gather(0, rows0, sg0)
        start_ef(off, efb0, se0)

        @pl.loop(0, NCHUNK)
        def _inner(jc):
            goff = pl.multiple_of(off + jc * CH1, 8)
            even = jc % 2 == 0

            @pl.when((jc + 1 < NCHUNK) & even)
            def _s1():
                start_gather(jc + 1, rows1, sg1)
                start_ef(goff + CH1, efb1, se1)

            @pl.when((jc + 1 < NCHUNK) & (~even))
            def _s0():
                start_gather(jc + 1, rows0, sg0)
                start_ef(goff + CH1, efb0, se0)

            @pl.when(even)
            def _c0():
                wait_g(rows0, sg0)
                if with_ef:
                    wait_e(efb0, se0)
                @pl.when(jc >= 2)
                def _wb():
                    wait_w(msg0, sw0)
                compute(rows0, efb0, msg0)
                start_w(msg0, goff, sw0)

            @pl.when(~even)
            def _c1():
                wait_g(rows1, sg1)
                if with_ef:
                    wait_e(efb1, se1)
                @pl.when(jc >= 2)
                def _wb():
                    wait_w(msg1, sw1)
                compute(rows1, efb1, msg1)
                start_w(msg1, goff, sw1)

        wait_w(msg1, sw1)
        wait_w(msg0, sw0)


def _sc_msg(h, ef, idx, with_ef, out_1d=False):
    mesh = plsc.VectorSubcoreMesh(core_axis_name="c", subcore_axis_name="s")
    sems = [pltpu.SemaphoreType.DMA] * 6
    oshape = (jax.ShapeDtypeStruct((E * H,), jnp.float32) if out_1d
              else jax.ShapeDtypeStruct((E, H), jnp.float32))
    mshape = ((CH1 * H,) if out_1d else (CH1, H))
    f = pl.kernel(
        functools.partial(_sc_msg_body, with_ef=with_ef, out_1d=out_1d),
        out_type=oshape,
        mesh=mesh,
        scratch_types=[pltpu.VMEM((KC1,), jnp.int32),
                       pltpu.VMEM((CH1, H2), jnp.float32),
                       pltpu.VMEM((CH1, H2), jnp.float32),
                       pltpu.VMEM((CH1, H), jnp.float32),
                       pltpu.VMEM((CH1, H), jnp.float32),
                       pltpu.VMEM(mshape, jnp.float32),
                       pltpu.VMEM(mshape, jnp.float32)] + sems,
    )
    return f(h, ef, idx)


def _sc_pair_body(h_hbm, src_hbm, dst_hbm, hs_hbm, hd_hbm, sidx,
                  rows0, rows1, msg0, msg1, sg0, sg1, sw0, sw1):
    c = lax.axis_index("c")
    sub = lax.axis_index("s")
    e0 = sub * EPT
    NCHUNK = KC1 // CH1

    def wait_g(rows, sem):
        pltpu.make_async_copy(h_hbm.at[sidx.at[pl.ds(0, CH1)]], rows,
                              sem).wait()

    def run(idx_hbm, out_hbm):
        def wait_w(msg, sem):
            pltpu.make_async_copy(msg, out_hbm.at[pl.ds(0, CH1)], sem).wait()

        def compute(rows, msg):
            @pl.loop(0, CH1)
            def _cmp(i):
                for j in range(H // L):
                    msg[i, pl.ds(j * L, L)] = rows[i, pl.ds(j * L, L)]

        @pl.loop(0, EPT // KC1)
        def _outer(ko):
            off = e0 + ko * KC1
            pltpu.sync_copy(idx_hbm.at[pl.ds(off, KC1)], sidx)
            pltpu.async_copy(h_hbm.at[sidx.at[pl.ds(0, CH1)]], rows0, sg0)

            @pl.loop(0, NCHUNK)
            def _inner(jc):
                goff = pl.multiple_of(off + jc * CH1, 8)
                even = jc % 2 == 0

                @pl.when((jc + 1 < NCHUNK) & even)
                def _s1():
                    pltpu.async_copy(
                        h_hbm.at[sidx.at[pl.ds((jc + 1) * CH1, CH1)]],
                        rows1, sg1)

                @pl.when((jc + 1 < NCHUNK) & (~even))
                def _s0():
                    pltpu.async_copy(
                        h_hbm.at[sidx.at[pl.ds((jc + 1) * CH1, CH1)]],
                        rows0, sg0)

                @pl.when(even)
                def _c0():
                    wait_g(rows0, sg0)
                    @pl.when(jc >= 2)
                    def _wb():
                        wait_w(msg0, sw0)
                    compute(rows0, msg0)
                    pltpu.async_copy(msg0, out_hbm.at[pl.ds(goff, CH1)], sw0)

                @pl.when(~even)
                def _c1():
                    wait_g(rows1, sg1)
                    @pl.when(jc >= 2)
                    def _wb():
                        wait_w(msg1, sw1)
                    compute(rows1, msg1)
                    pltpu.async_copy(msg1, out_hbm.at[pl.ds(goff, CH1)], sw1)

            wait_w(msg1, sw1)
            wait_w(msg0, sw0)

    @pl.when(c == 0)
    def _c0():
        run(src_hbm, hs_hbm)

    @pl.when(c == 1)
    def _c1():
        run(dst_hbm, hd_hbm)


def _sc_pair(h, src, dst):
    mesh = plsc.VectorSubcoreMesh(core_axis_name="c", subcore_axis_name="s")
    f = pl.kernel(
        _sc_pair_body,
        out_type=[jax.ShapeDtypeStruct((E, H), jnp.float32),
                  jax.ShapeDtypeStruct((E, H), jnp.float32)],
        mesh=mesh,
        scratch_types=[pltpu.VMEM((KC1,), jnp.int32),
                       pltpu.VMEM((CH1, H2), jnp.float32),
                       pltpu.VMEM((CH1, H2), jnp.float32),
                       pltpu.VMEM((CH1, H), jnp.float32),
                       pltpu.VMEM((CH1, H), jnp.float32),
                       pltpu.SemaphoreType.DMA, pltpu.SemaphoreType.DMA,
                       pltpu.SemaphoreType.DMA, pltpu.SemaphoreType.DMA],
    )
    return f(h, src, dst)


def _sc_seg_body(msg_hbm, dst_hbm, out_hbm, didx, lidx, mb0, mb1, mbc,
                 accum, sr0, sr1):
    c = lax.axis_index("c")
    sub = lax.axis_index("s")
    base = c * NHALF
    NCHUNK = KC2 // CH2

    @pl.loop(0, CH2)
    def _zr(i):
        for j in range(H // L):
            mbc[i, pl.ds(j * L, L)] = jnp.zeros((L,), jnp.float32)

    zpt = ACC // NS
    z0 = sub * zpt
    @pl.loop(0, zpt // CH2)
    def _zc(k):
        pltpu.sync_copy(mbc, accum.at[pl.ds(z0 + k * CH2, CH2)])
    pltpu.sync_copy(mbc.at[pl.ds(0, zpt % CH2)],
                    accum.at[pl.ds(z0 + (zpt // CH2) * CH2, zpt % CH2)])

    plsc.subcore_barrier()

    def wait_read(ref, sem):
        pltpu.make_async_copy(msg_hbm.at[pl.ds(0, CH2 * H)], ref, sem).wait()

    def compact_scatter(mb, jc):
        @pl.loop(0, CH2)
        def _cp(i):
            for j in range(H // L):
                mbc[i, pl.ds(j * L, L)] = mb[pl.ds(i * H + j * L, L)]
        pltpu.sync_copy(mbc, accum.at[lidx.at[jc]], add=True)

    e0 = sub * EPT
    @pl.loop(0, EPT // KC2)
    def _outer(ko):
        off = e0 + ko * KC2
        pltpu.sync_copy(dst_hbm.at[pl.ds(off, KC2)], didx)

        @pl.loop(0, NCHUNK)
        def _lix(r):
            for j in range(CH2 // L):
                v = didx[pl.ds(r * CH2 + j * L, L)] - base
                oob = (v < 0) | (v >= NHALF)
                lidx[r, pl.ds(j * L, L)] = jnp.where(oob, DUM, v)

        pltpu.async_copy(msg_hbm.at[pl.ds(pl.multiple_of(off * H, 8),
                                          CH2 * H)], mb0, sr0)

        @pl.loop(0, NCHUNK)
        def _inner(jc):
            goff = pl.multiple_of((off + jc * CH2) * H, 8)
            even = jc % 2 == 0

            @pl.when((jc + 1 < NCHUNK) & even)
            def _s1():
                pltpu.async_copy(msg_hbm.at[pl.ds(goff + CH2 * H, CH2 * H)],
                                 mb1, sr1)

            @pl.when((jc + 1 < NCHUNK) & (~even))
            def _s0():
                pltpu.async_copy(msg_hbm.at[pl.ds(goff + CH2 * H, CH2 * H)],
                                 mb0, sr0)

            @pl.when(even)
            def _c0():
                wait_read(mb0, sr0)
                compact_scatter(mb0, jc)

            @pl.when(~even)
            def _c1():
                wait_read(mb1, sr1)
                compact_scatter(mb1, jc)

    plsc.subcore_barrier()

    per_tec = NHALF // NS
    a0 = sub * per_tec
    @pl.loop(0, per_tec // CH2)
    def _co(k):
        a = pl.multiple_of(a0 + k * CH2, 8)
        pltpu.sync_copy(accum.at[pl.ds(a, CH2)], mbc)
        pltpu.sync_copy(mbc, out_hbm.at[pl.ds(pl.multiple_of(
            base + a, 8), CH2)])

    rem = per_tec % CH2
    a = pl.multiple_of(a0 + (per_tec // CH2) * CH2, 8)
    pltpu.sync_copy(accum.at[pl.ds(a, rem)], mbc.at[pl.ds(0, rem)])
    pltpu.sync_copy(mbc.at[pl.ds(0, rem)],
                    out_hbm.at[pl.ds(pl.multiple_of(base + a, 8), rem)])


def _sc_seg(msg, dst):
    mesh = plsc.VectorSubcoreMesh(core_axis_name="c", subcore_axis_name="s")
    f = pl.kernel(
        _sc_seg_body,
        out_type=jax.ShapeDtypeStruct((NP, H), jnp.float32),
        mesh=mesh,
        scratch_types=[pltpu.VMEM((KC2,), jnp.int32),
                       pltpu.VMEM((KC2 // CH2, CH2), jnp.int32),
                       pltpu.VMEM((CH2 * H,), jnp.float32),
                       pltpu.VMEM((CH2 * H,), jnp.float32),
                       pltpu.VMEM((CH2, H), jnp.float32),
                       pltpu.VMEM_SHARED((ACC, H), jnp.float32),
                       pltpu.SemaphoreType.DMA, pltpu.SemaphoreType.DMA],
        compiler_params=pltpu.CompilerParams(use_tc_tiling_on_sc=False),
    )
    return f(msg, dst)
def kernel(x, edge_index, edge_attr, batch, params):
    p = params
    src = edge_index[0]
    dst = edge_index[1]

    xp = jnp.pad(x, ((0, NP - N), (0, 5)))
    eap = jnp.pad(edge_attr, ((0, 0), (0, 5)))
    batch_r = jnp.pad(batch, (0, NP - N), constant_values=NG).reshape(
        NP // BN, 1, BN)
    src_r = src.reshape(E // BE, 1, BE)

    r1 = lambda a: a.reshape(1, -1)
    padw = lambda a: jnp.pad(a, ((0, 0), (0, H2 - H)))
    ne_w1 = jnp.pad(p['ne_w1'], ((0, 5), (0, 0)))
    ee_w1 = jnp.pad(p['ee_w1'], ((0, 5), (0, 0)))

    h = _encoder(xp, ne_w1, r1(p['ne_b1']), padw(p['ne_w2']),
                 padw(r1(p['ne_b2'])), padw(r1(p['ne_g'])),
                 padw(r1(p['ne_b'])), BN, wide=True)
    ef = _encoder(eap, ee_w1, r1(p['ee_b1']), p['ee_w2'], r1(p['ee_b2']),
                  r1(p['ee_g']), r1(p['ee_b']), BE, wide=False)

    for i in range(2):
        q = p['gin%d' % i]
        msg = _sc_msg(h, ef, src, with_ef=True, out_1d=True)
        aggr = _sc_seg(msg, dst)
        h = _gin_mlp(h, aggr, q['eps'].reshape(1, 1), q['w1'], r1(q['b1']),
                     padw(q['w2']), padw(r1(q['b2'])), padw(r1(q['g'])),
                     padw(r1(q['b'])), relu_out=(i == 0))

    g, starts = _pool(h, batch_r, p['gp_w'], r1(p['gp_b']), r1(p['gp_g']),
                      r1(p['gp_beta']))

    hs, hd = _sc_pair(h, src, dst)

    w1 = p['ep_w1']
    o = _edge_mlp(hs, hd, ef, src_r, starts, g,
                  w1[0:H], w1[H:2 * H], w1[2 * H:3 * H], w1[3 * H:4 * H],
                  r1(p['ep_b1']), p['ep_w2'], r1(p['ep_b2']), p['ep_w3'],
                  p['ep_b3'].reshape(1, 1))
    return o
